# Initial kernel scaffold; baseline (speedup 1.0000x reference)
import jax
import jax.numpy as jnp
from jax.experimental import pallas as pl


def kernel(logits, temperatures, top_ps):
    raise NotImplementedError("write your pallas kernel here")


# speedup vs baseline: 26.4015x; 26.4015x over previous
import functools

import jax
import jax.numpy as jnp
from jax import lax
from jax.experimental import pallas as pl
from jax.experimental.pallas import tpu as pltpu
from jax.experimental.pallas import tpu_sc as plsc

B = 64
V = 100000
L = 16
NW = 32
ROWS_PER_W = B // NW
CHUNK = 10000
N_CHUNKS = V // CHUNK
VREGS = V // L
CREGS = CHUNK // L
H1BITS, H2BITS, H3BITS = 11, 11, 9
H1, H2, H3 = 1 << H1BITS, 1 << H2BITS, 1 << H3BITS
I32MAX = jnp.int32(2**31 - 1)

_CONSTS = {}


def _noise_flat():
    if "noise" not in _CONSTS:
        n = jax.random.exponential(jax.random.key(42), (B, V), dtype=jnp.float32)
        _CONSTS["noise"] = jnp.maximum(n, 1e-10).reshape(-1)
    return _CONSTS["noise"]


def _scan_hist(href, nbins, lanes, a0, p):
    nblk = nbins // L

    def body(j, st):
        a, bstar, found = st
        i = nblk - 1 - j
        blk = href[pl.ds(i * L, L)]
        rev = lax.rev(blk, (0,))
        incl = plsc.cumsum(rev)
        elig = (a + incl) > p
        bins_desc = i * L + (L - 1) - lanes
        cand = jnp.where(elig, bins_desc, jnp.int32(-1))
        blk_b = jnp.max(cand)
        bfound = blk_b >= 0
        inclmin = jnp.min(jnp.where(elig, incl, jnp.float32(jnp.inf)))
        safe_b = jnp.maximum(blk_b, 0)
        hb = jnp.max(plsc.load_gather(href, [jnp.full((L,), safe_b, jnp.int32)]))
        blksum = jnp.sum(blk)
        a_new = jnp.where(found, a,
                          jnp.where(bfound, a + inclmin - hb, a + blksum))
        b_new = jnp.where(found, bstar, jnp.where(bfound, blk_b, bstar))
        return a_new, b_new, found | bfound

    return lax.fori_loop(0, nblk, body, (a0, jnp.int32(0), False))


def _sc_body(logits_hbm, temps_hbm, topp_hbm, noise_hbm, out_hbm,
             qbuf, nbuf, h1, h2, h3, tbuf, pbuf, obuf):
    cid = lax.axis_index("c")
    sid = lax.axis_index("s")
    wid = sid * 2 + cid
    lanes = lax.iota(jnp.int32, 16)
    toks = []

    for rslot in range(ROWS_PER_W):
        row = wid * ROWS_PER_W + rslot
        pltpu.sync_copy(logits_hbm.at[pl.ds(row * V, V)], qbuf)
        pltpu.sync_copy(temps_hbm.at[row], tbuf)
        pltpu.sync_copy(topp_hbm.at[row], pbuf)
        t_vec = tbuf[...]
        t_safe = jnp.maximum(t_vec, jnp.float32(1e-5))
        p = jnp.max(pbuf[...])

        def pass_a(i, carry):
            bl, bi = carry
            v = qbuf[pl.ds(i * L, L)]
            upd = v > bl
            return jnp.where(upd, v, bl), jnp.where(upd, i * L + lanes, bi)

        bl, bi = lax.fori_loop(
            0, VREGS, pass_a,
            (jnp.full((L,), -jnp.inf, jnp.float32), jnp.zeros((L,), jnp.int32)))
        m_l = jnp.max(bl)
        greedy = jnp.min(jnp.where(bl == m_l, bi, I32MAX))
        mx_vec = jnp.full((L,), m_l) / t_safe

        def pass_b(i, zacc):
            v = qbuf[pl.ds(i * L, L)]
            e = jnp.exp(v / t_safe - mx_vec)
            qbuf[pl.ds(i * L, L)] = e
            return zacc + e

        zacc = lax.fori_loop(0, VREGS, pass_b, jnp.zeros((L,), jnp.float32))
        z_vec = jnp.full((L,), jnp.sum(zacc))

        def clr(href, nbins):
            def cb(i, _):
                href[pl.ds(i * L, L)] = jnp.zeros((L,), jnp.float32)
                return 0
            lax.fori_loop(0, nbins // L, cb, 0)

        clr(h1, H1)
        clr(h2, H2)
        clr(h3, H3)

        def pass_c(i, carry):
            bq, bi2 = carry
            q = qbuf[pl.ds(i * L, L)] / z_vec
            qbuf[pl.ds(i * L, L)] = q
            key = plsc.bitcast(q, jnp.int32)
            plsc.addupdate_scatter(h1, [lax.shift_right_logical(key, H2BITS + H3BITS)], q)
            upd = q > bq
            return jnp.where(upd, q, bq), jnp.where(upd, i * L + lanes, bi2)

        bq, bi2 = lax.fori_loop(
            0, VREGS, pass_c,
            (jnp.full((L,), -1.0, jnp.float32), jnp.zeros((L,), jnp.int32)))
        top1 = jnp.min(jnp.where(bq == jnp.max(bq), bi2, I32MAX))

        a1, b1, f1 = _scan_hist(h1, H1, lanes, jnp.float32(0.0), p)

        b1sel = jnp.where(f1, b1, jnp.int32(-1))
        b1v = jnp.full((L,), b1sel)

        def pass_d(i, _):
            q = qbuf[pl.ds(i * L, L)]
            key = plsc.bitcast(q, jnp.int32)
            m2 = lax.shift_right_logical(key, H2BITS + H3BITS) == b1v
            bin2 = lax.shift_right_logical(key, H3BITS) & jnp.int32(H2 - 1)
            plsc.addupdate_scatter(h2, [bin2], q, mask=m2)
            return 0

        lax.fori_loop(0, VREGS, pass_d, 0)
        a2, b2, f2 = _scan_hist(h2, H2, lanes, a1, p)

        pre23 = jnp.where(f1 & f2, b1 * H2 + b2, jnp.int32(-1))
        pre23v = jnp.full((L,), pre23)

        def pass_e(i, _):
            q = qbuf[pl.ds(i * L, L)]
            key = plsc.bitcast(q, jnp.int32)
            m3 = lax.shift_right_logical(key, H3BITS) == pre23v
            bin3 = key & jnp.int32(H3 - 1)
            plsc.addupdate_scatter(h3, [bin3], q, mask=m3)
            return 0

        lax.fori_loop(0, VREGS, pass_e, 0)
        a3, b3, f3 = _scan_hist(h3, H3, lanes, a2, p)

        tau_key = jnp.where(f1, (b1 << (H2BITS + H3BITS)) | (b2 << H3BITS) | b3,
                            jnp.int32(-1))
        tau_v = jnp.full((L,), tau_key)
        tau_f = plsc.bitcast(tau_v, jnp.float32)
        a_v = jnp.full((L,), a3)
        p_v = jnp.full((L,), p)
        top1_v = jnp.full((L,), top1)

        def run_chunk(c, carry0):
            pltpu.sync_copy(noise_hbm.at[pl.ds(row * V + c * CHUNK, CHUNK)], nbuf)

            def pass_f(i, carry):
                bg, bwi, ceq = carry
                gidx = c * CHUNK + i * L
                q = qbuf[pl.ds(gidx, L)]
                nz = nbuf[pl.ds(i * L, L)]
                key = plsc.bitcast(q, jnp.int32)
                gt = key > tau_v
                eq = key == tau_v
                kranks = ceq + plsc.cumsum(eq.astype(jnp.int32))
                kepteq = eq & ((a_v + kranks.astype(jnp.float32) * tau_f) <= p_v)
                idxv = gidx + lanes
                kept = gt | kepteq | (idxv == top1_v)
                gm = jnp.where(kept, q / nz, jnp.float32(-1.0))
                upd = gm > bg
                return (jnp.where(upd, gm, bg), jnp.where(upd, idxv, bwi),
                        ceq + jnp.sum(eq.astype(jnp.int32)))

            return lax.fori_loop(0, CREGS, pass_f, carry0)

        bg, bwi, _ = lax.fori_loop(
            0, N_CHUNKS, run_chunk,
            (jnp.full((L,), -2.0, jnp.float32), jnp.zeros((L,), jnp.int32),
             jnp.int32(0)))
        winner = jnp.min(jnp.where(bg == jnp.max(bg), bwi, I32MAX))

        tok = jnp.where(jnp.max(t_vec) <= jnp.float32(1e-10), greedy, winner)
        toks.append(tok)

    ovec = jnp.where(lanes == 0, toks[0],
                     jnp.where(lanes == 1, toks[1], jnp.int32(0)))
    obuf[...] = ovec
    pltpu.sync_copy(obuf, out_hbm.at[wid])


@functools.partial(jax.jit, static_argnames=())
def _sampler(logits_flat, temps16, topp16, noise_flat):
    mesh = plsc.VectorSubcoreMesh(core_axis_name="c", subcore_axis_name="s",
                                  num_cores=2, num_subcores=16)
    k = pl.kernel(
        _sc_body,
        out_type=jax.ShapeDtypeStruct((NW, L), jnp.int32),
        mesh=mesh,
        compiler_params=pltpu.CompilerParams(needs_layout_passes=False),
        scratch_types=[
            pltpu.VMEM((V,), jnp.float32),
            pltpu.VMEM((CHUNK,), jnp.float32),
            pltpu.VMEM((H1,), jnp.float32),
            pltpu.VMEM((H2,), jnp.float32),
            pltpu.VMEM((H3,), jnp.float32),
            pltpu.VMEM((L,), jnp.float32),
            pltpu.VMEM((L,), jnp.float32),
            pltpu.VMEM((L,), jnp.int32),
        ],
    )
    return k(logits_flat, temps16, topp16, noise_flat)


def kernel(logits, temperatures, top_ps):
    logits = logits.astype(jnp.float32)
    noise = _noise_flat()
    temps16 = jnp.tile(temperatures.astype(jnp.float32)[:, None], (1, L))
    topp16 = jnp.tile(top_ps.astype(jnp.float32)[:, None], (1, L))
    out = _sampler(logits.reshape(-1), temps16, topp16, noise)
    return out[:, :ROWS_PER_W].reshape(-1)


# speedup vs baseline: 31.1529x; 1.1800x over previous
import functools

import jax
import jax.numpy as jnp
from jax import lax
from jax.experimental import pallas as pl
from jax.experimental.pallas import tpu as pltpu
from jax.experimental.pallas import tpu_sc as plsc

B = 64
V = 100000
L = 16
NW = 32
ROWS_PER_W = B // NW
CHUNK = 4000
N_CHUNKS = V // CHUNK
VREGS = V // L
CREGS = CHUNK // L
H1BITS, H2BITS, H3BITS = 11, 11, 9
H1, H2, H3 = 1 << H1BITS, 1 << H2BITS, 1 << H3BITS
I32MAX = jnp.int32(2**31 - 1)

_CONSTS = {}


def _noise_flat():
    if "noise" not in _CONSTS:
        n = jax.random.exponential(jax.random.key(42), (B, V), dtype=jnp.float32)
        _CONSTS["noise"] = jnp.maximum(n, 1e-10).reshape(-1)
    return _CONSTS["noise"]


def _scan_hist(href, nbins, lanes, a0, p):
    nblk = nbins // L

    def body(j, st):
        a, bstar, found = st
        i = nblk - 1 - j
        blk = href[pl.ds(i * L, L)]
        rev = lax.rev(blk, (0,))
        incl = plsc.cumsum(rev)
        elig = (a + incl) > p
        bins_desc = i * L + (L - 1) - lanes
        cand = jnp.where(elig, bins_desc, jnp.int32(-1))
        blk_b = jnp.max(cand)
        bfound = blk_b >= 0
        inclmin = jnp.min(jnp.where(elig, incl, jnp.float32(jnp.inf)))
        safe_b = jnp.maximum(blk_b, 0)
        hb = jnp.max(plsc.load_gather(href, [jnp.full((L,), safe_b, jnp.int32)]))
        blksum = jnp.sum(blk)
        a_new = jnp.where(found, a,
                          jnp.where(bfound, a + inclmin - hb, a + blksum))
        b_new = jnp.where(found, bstar, jnp.where(bfound, blk_b, bstar))
        return a_new, b_new, found | bfound

    return lax.fori_loop(0, nblk, body, (a0, jnp.int32(0), False))


def _sc_body(logits_hbm, temps_hbm, topp_hbm, noise_hbm, out_hbm,
             qbuf, nbuf0, nbuf1, h1, h2, h3, h3c, tbuf, pbuf, obuf, sem0, sem1):
    cid = lax.axis_index("c")
    sid = lax.axis_index("s")
    wid = sid * 2 + cid
    lanes = lax.iota(jnp.int32, 16)
    nsems = [sem0, sem1]
    toks = []

    for rslot in range(ROWS_PER_W):
        row = wid * ROWS_PER_W + rslot
        pltpu.sync_copy(logits_hbm.at[pl.ds(row * V, V)], qbuf)
        pltpu.sync_copy(temps_hbm.at[row], tbuf)
        pltpu.sync_copy(topp_hbm.at[row], pbuf)

        nbufs = [nbuf0, nbuf1]

        def noise_copy(c):
            return pltpu.make_async_copy(
                noise_hbm.at[pl.ds(row * V + c * CHUNK, CHUNK)],
                nbufs[c % 2], nsems[c % 2])

        noise_copy(0).start()

        t_vec = tbuf[...]
        t_safe = jnp.maximum(t_vec, jnp.float32(1e-5))
        p = jnp.max(pbuf[...])

        def pass_a(i, carry):
            bl, bi = carry
            for u in range(2):
                o = (2 * i + u) * L
                v = qbuf[pl.ds(o, L)]
                upd = v > bl
                bl = jnp.where(upd, v, bl)
                bi = jnp.where(upd, o + lanes, bi)
            return bl, bi

        bl, bi = lax.fori_loop(
            0, VREGS // 2, pass_a,
            (jnp.full((L,), -jnp.inf, jnp.float32), jnp.zeros((L,), jnp.int32)))
        m_l = jnp.max(bl)
        greedy = jnp.min(jnp.where(bl == m_l, bi, I32MAX))
        mx_vec = jnp.full((L,), m_l) / t_safe

        def pass_b(i, zacc):
            for u in range(2):
                o = (2 * i + u) * L
                e = jnp.exp(qbuf[pl.ds(o, L)] / t_safe - mx_vec)
                qbuf[pl.ds(o, L)] = e
                zacc = zacc + e
            return zacc

        zacc = lax.fori_loop(0, VREGS // 2, pass_b, jnp.zeros((L,), jnp.float32))
        z_vec = jnp.full((L,), jnp.sum(zacc))

        def clr(href, nbins, zero):
            def cb(i, _):
                href[pl.ds(i * L, L)] = zero
                return 0
            lax.fori_loop(0, nbins // L, cb, 0)

        clr(h1, H1, jnp.zeros((L,), jnp.float32))
        clr(h2, H2, jnp.zeros((L,), jnp.float32))
        clr(h3, H3, jnp.zeros((L,), jnp.float32))
        clr(h3c, H3, jnp.zeros((L,), jnp.int32))

        def pass_c(i, carry):
            bq, bi2 = carry
            for u in range(2):
                o = (2 * i + u) * L
                q = qbuf[pl.ds(o, L)] / z_vec
                qbuf[pl.ds(o, L)] = q
                key = plsc.bitcast(q, jnp.int32)
                plsc.addupdate_scatter(
                    h1, [lax.shift_right_logical(key, H2BITS + H3BITS)], q)
                upd = q > bq
                bq = jnp.where(upd, q, bq)
                bi2 = jnp.where(upd, o + lanes, bi2)
            return bq, bi2

        bq, bi2 = lax.fori_loop(
            0, VREGS // 2, pass_c,
            (jnp.full((L,), -1.0, jnp.float32), jnp.zeros((L,), jnp.int32)))
        top1 = jnp.min(jnp.where(bq == jnp.max(bq), bi2, I32MAX))

        a1, b1, f1 = _scan_hist(h1, H1, lanes, jnp.float32(0.0), p)

        b1v = jnp.full((L,), jnp.where(f1, b1, jnp.int32(-1)))

        def pass_d(i, _):
            for u in range(2):
                o = (2 * i + u) * L
                key = plsc.bitcast(qbuf[pl.ds(o, L)], jnp.int32)
                m2 = lax.shift_right_logical(key, H2BITS + H3BITS) == b1v
                bin2 = lax.shift_right_logical(key, H3BITS) & jnp.int32(H2 - 1)
                plsc.addupdate_scatter(h2, [bin2], qbuf[pl.ds(o, L)], mask=m2)
            return 0

        lax.fori_loop(0, VREGS // 2, pass_d, 0)
        a2, b2, f2 = _scan_hist(h2, H2, lanes, a1, p)

        pre23v = jnp.full((L,), jnp.where(f1 & f2, b1 * H2 + b2, jnp.int32(-1)))
        ones_i = jnp.full((L,), 1, jnp.int32)

        def pass_e(i, _):
            for u in range(2):
                o = (2 * i + u) * L
                q = qbuf[pl.ds(o, L)]
                key = plsc.bitcast(q, jnp.int32)
                m3 = lax.shift_right_logical(key, H3BITS) == pre23v
                bin3 = key & jnp.int32(H3 - 1)
                plsc.addupdate_scatter(h3, [bin3], q, mask=m3)
                plsc.addupdate_scatter(h3c, [bin3], ones_i, mask=m3)
            return 0

        lax.fori_loop(0, VREGS // 2, pass_e, 0)
        a3, b3, f3 = _scan_hist(h3, H3, lanes, a2, p)

        tau_key = jnp.where(f1, (b1 << (H2BITS + H3BITS)) | (b2 << H3BITS) | b3,
                            jnp.int32(-1))
        tau_v = jnp.full((L,), tau_key)
        tau_f = plsc.bitcast(tau_v, jnp.float32)
        tau_s = jnp.max(tau_f)
        n_eq = jnp.max(plsc.load_gather(
            h3c, [jnp.full((L,), jnp.maximum(b3, 0), jnp.int32)]))

        def bs_body(_, st):
            lo, hi = st
            mid = lax.shift_right_logical(lo + hi + 1, 1)
            c = (a3 + mid.astype(jnp.float32) * tau_s) <= p
            return jnp.where(c, mid, lo), jnp.where(c, hi, mid - 1)

        kstar, _ = lax.fori_loop(0, 17, bs_body, (jnp.int32(0), n_eq))

        def find_cut(_):
            def fc_body(i, st):
                cnt, cut = st
                key = plsc.bitcast(qbuf[pl.ds(i * L, L)], jnp.int32)
                eq = key == tau_v
                kr = cnt + plsc.cumsum(eq.astype(jnp.int32))
                hit = eq & (kr == kstar)
                cut = jnp.maximum(cut, jnp.max(jnp.where(hit, i * L + lanes,
                                                         jnp.int32(-1))))
                cnt = cnt + plsc.all_reduce_population_count(eq)
                return cnt, cut

            _, cut = lax.fori_loop(0, VREGS, fc_body,
                                   (jnp.zeros((L,), jnp.int32), jnp.int32(-1)))
            return cut

        cut = lax.cond(kstar >= n_eq, lambda _: I32MAX,
                       lambda _: lax.cond(kstar == 0, lambda __: jnp.int32(-1),
                                          find_cut, _),
                       0)
        cut_v = jnp.full((L,), cut)
        a_v = jnp.full((L,), a3)
        top1_v = jnp.full((L,), top1)

        bg = jnp.full((L,), -2.0, jnp.float32)
        bwi = jnp.zeros((L,), jnp.int32)
        for c in range(N_CHUNKS):
            if c + 1 < N_CHUNKS:
                noise_copy(c + 1).start()
            noise_copy(c).wait()
            cbase = c * CHUNK
            nb = nbufs[c % 2]

            def pass_f(i, carry, cbase=cbase, nb=nb):
                bg, bwi = carry
                for u in range(2):
                    o = (2 * i + u) * L
                    q = qbuf[pl.ds(cbase + o, L)]
                    nz = nb[pl.ds(o, L)]
                    key = plsc.bitcast(q, jnp.int32)
                    idxv = cbase + o + lanes
                    kept = ((key > tau_v)
                            | ((key == tau_v) & (idxv <= cut_v))
                            | (idxv == top1_v))
                    gm = jnp.where(kept, q / nz, jnp.float32(-1.0))
                    upd = gm > bg
                    bg = jnp.where(upd, gm, bg)
                    bwi = jnp.where(upd, idxv, bwi)
                return bg, bwi

            bg, bwi = lax.fori_loop(0, CREGS // 2, pass_f, (bg, bwi))

        winner = jnp.min(jnp.where(bg == jnp.max(bg), bwi, I32MAX))
        tok = jnp.where(jnp.max(t_vec) <= jnp.float32(1e-10), greedy, winner)
        toks.append(tok)

    ovec = jnp.where(lanes == 0, toks[0],
                     jnp.where(lanes == 1, toks[1], jnp.int32(0)))
    obuf[...] = ovec
    pltpu.sync_copy(obuf, out_hbm.at[wid])


@jax.jit
def _sampler(logits_flat, temps16, topp16, noise_flat):
    mesh = plsc.VectorSubcoreMesh(core_axis_name="c", subcore_axis_name="s",
                                  num_cores=2, num_subcores=16)
    k = pl.kernel(
        _sc_body,
        out_type=jax.ShapeDtypeStruct((NW, L), jnp.int32),
        mesh=mesh,
        compiler_params=pltpu.CompilerParams(needs_layout_passes=False),
        scratch_types=[
            pltpu.VMEM((V,), jnp.float32),
            pltpu.VMEM((CHUNK,), jnp.float32),
            pltpu.VMEM((CHUNK,), jnp.float32),
            pltpu.VMEM((H1,), jnp.float32),
            pltpu.VMEM((H2,), jnp.float32),
            pltpu.VMEM((H3,), jnp.float32),
            pltpu.VMEM((H3,), jnp.int32),
            pltpu.VMEM((L,), jnp.float32),
            pltpu.VMEM((L,), jnp.float32),
            pltpu.VMEM((L,), jnp.int32),
            pltpu.SemaphoreType.DMA,
            pltpu.SemaphoreType.DMA,
        ],
    )
    return k(logits_flat, temps16, topp16, noise_flat)


def kernel(logits, temperatures, top_ps):
    logits = logits.astype(jnp.float32)
    noise = _noise_flat()
    temps16 = jnp.tile(temperatures.astype(jnp.float32)[:, None], (1, L))
    topp16 = jnp.tile(top_ps.astype(jnp.float32)[:, None], (1, L))
    out = _sampler(logits.reshape(-1), temps16, topp16, noise)
    return out[:, :ROWS_PER_W].reshape(-1)


# speedup vs baseline: 33.2413x; 1.0670x over previous
import functools

import jax
import jax.numpy as jnp
from jax import lax
from jax.experimental import pallas as pl
from jax.experimental.pallas import tpu as pltpu
from jax.experimental.pallas import tpu_sc as plsc

B = 64
V = 100000
L = 16
NW = 32
ROWS_PER_W = B // NW
CHUNK = 2000
N_CHUNKS = V // CHUNK
VREGS = V // L
CREGS = CHUNK // L
H1BITS, H2BITS, H3BITS = 11, 11, 9
H1, H2, H3 = 1 << H1BITS, 1 << H2BITS, 1 << H3BITS
I32MAX = jnp.int32(2**31 - 1)

_CONSTS = {}


def _noise_flat():
    if "noise" not in _CONSTS:
        n = jax.random.exponential(jax.random.key(42), (B, V), dtype=jnp.float32)
        _CONSTS["noise"] = jnp.maximum(n, 1e-10).reshape(-1)
    return _CONSTS["noise"]


def _scan_hist(href, nbins, lanes, a0, p):
    nblk = nbins // L

    def body(j, st):
        a, bstar, found = st
        i = nblk - 1 - j
        blk = href[pl.ds(i * L, L)]
        rev = lax.rev(blk, (0,))
        incl = plsc.cumsum(rev)
        elig = (a + incl) > p
        bins_desc = i * L + (L - 1) - lanes
        cand = jnp.where(elig, bins_desc, jnp.int32(-1))
        blk_b = jnp.max(cand)
        bfound = blk_b >= 0
        inclmin = jnp.min(jnp.where(elig, incl, jnp.float32(jnp.inf)))
        safe_b = jnp.maximum(blk_b, 0)
        hb = jnp.max(plsc.load_gather(href, [jnp.full((L,), safe_b, jnp.int32)]))
        blksum = jnp.sum(blk)
        a_new = jnp.where(found, a,
                          jnp.where(bfound, a + inclmin - hb, a + blksum))
        b_new = jnp.where(found, bstar, jnp.where(bfound, blk_b, bstar))
        return a_new, b_new, found | bfound

    return lax.fori_loop(0, nblk, body, (a0, jnp.int32(0), False))


def _sc_body(logits_hbm, temps_hbm, topp_hbm, noise_hbm, out_hbm,
             qbuf, nbuf0, nbuf1, h1, h2, h3, h3c, tbuf, pbuf, obuf, sem0, sem1):
    cid = lax.axis_index("c")
    sid = lax.axis_index("s")
    wid = sid * 2 + cid
    lanes = lax.iota(jnp.int32, 16)
    nsems = [sem0, sem1]
    toks = []

    for rslot in range(ROWS_PER_W):
        row = wid * ROWS_PER_W + rslot
        pltpu.sync_copy(logits_hbm.at[pl.ds(row * V, V)], qbuf)
        pltpu.sync_copy(temps_hbm.at[row], tbuf)
        pltpu.sync_copy(topp_hbm.at[row], pbuf)

        nbufs = [nbuf0, nbuf1]

        def noise_copy(c):
            return pltpu.make_async_copy(
                noise_hbm.at[pl.ds(row * V + c * CHUNK, CHUNK)],
                nbufs[c % 2], nsems[c % 2])

        def noise_copy_dyn(c, nb, sem):
            return pltpu.make_async_copy(
                noise_hbm.at[pl.ds(row * V + c * CHUNK, CHUNK)], nb, sem)

        noise_copy(0).start()

        t_vec = tbuf[...]
        t_safe = jnp.maximum(t_vec, jnp.float32(1e-5))
        p = jnp.max(pbuf[...])

        @plsc.parallel_loop(0, VREGS, unroll=5, carry=(
            jnp.full((L,), -jnp.inf, jnp.float32), jnp.zeros((L,), jnp.int32)))
        def pass_a(i, carry):
            bl, bi = carry
            v = qbuf[pl.ds(i * L, L)]
            upd = v > bl
            return jnp.where(upd, v, bl), jnp.where(upd, i * L + lanes, bi)

        bl, bi = pass_a
        m_l = jnp.max(bl)
        greedy = jnp.min(jnp.where(bl == m_l, bi, I32MAX))
        mx_vec = jnp.full((L,), m_l) / t_safe

        @plsc.parallel_loop(0, VREGS, unroll=5,
                            carry=jnp.zeros((L,), jnp.float32))
        def pass_b(i, zacc):
            e = jnp.exp(qbuf[pl.ds(i * L, L)] / t_safe - mx_vec)
            qbuf[pl.ds(i * L, L)] = e
            return zacc + e

        zacc = pass_b
        z_vec = jnp.full((L,), jnp.sum(zacc))

        def clr(href, nbins, zero):
            @plsc.parallel_loop(0, nbins // L, unroll=4)
            def cb(i):
                href[pl.ds(i * L, L)] = zero

        clr(h1, H1, jnp.zeros((L,), jnp.float32))
        clr(h2, H2, jnp.zeros((L,), jnp.float32))
        clr(h3, H3, jnp.zeros((L,), jnp.float32))
        clr(h3c, H3, jnp.zeros((L,), jnp.int32))

        def pass_c(i, carry):
            bq, bi2 = carry
            for u in range(2):
                o = (2 * i + u) * L
                q = qbuf[pl.ds(o, L)] / z_vec
                qbuf[pl.ds(o, L)] = q
                key = plsc.bitcast(q, jnp.int32)
                plsc.addupdate_scatter(
                    h1, [lax.shift_right_logical(key, H2BITS + H3BITS)], q)
                upd = q > bq
                bq = jnp.where(upd, q, bq)
                bi2 = jnp.where(upd, o + lanes, bi2)
            return bq, bi2

        bq, bi2 = lax.fori_loop(
            0, VREGS // 2, pass_c,
            (jnp.full((L,), -1.0, jnp.float32), jnp.zeros((L,), jnp.int32)))
        top1 = jnp.min(jnp.where(bq == jnp.max(bq), bi2, I32MAX))

        a1, b1, f1 = _scan_hist(h1, H1, lanes, jnp.float32(0.0), p)

        b1v = jnp.full((L,), jnp.where(f1, b1, jnp.int32(-1)))

        def pass_d(i, _):
            for u in range(2):
                o = (2 * i + u) * L
                key = plsc.bitcast(qbuf[pl.ds(o, L)], jnp.int32)
                m2 = lax.shift_right_logical(key, H2BITS + H3BITS) == b1v
                bin2 = lax.shift_right_logical(key, H3BITS) & jnp.int32(H2 - 1)
                plsc.addupdate_scatter(h2, [bin2], qbuf[pl.ds(o, L)], mask=m2)
            return 0

        lax.fori_loop(0, VREGS // 2, pass_d, 0)
        a2, b2, f2 = _scan_hist(h2, H2, lanes, a1, p)

        pre23v = jnp.full((L,), jnp.where(f1 & f2, b1 * H2 + b2, jnp.int32(-1)))
        ones_i = jnp.full((L,), 1, jnp.int32)

        def pass_e(i, _):
            for u in range(2):
                o = (2 * i + u) * L
                q = qbuf[pl.ds(o, L)]
                key = plsc.bitcast(q, jnp.int32)
                m3 = lax.shift_right_logical(key, H3BITS) == pre23v
                bin3 = key & jnp.int32(H3 - 1)
                plsc.addupdate_scatter(h3, [bin3], q, mask=m3)
                plsc.addupdate_scatter(h3c, [bin3], ones_i, mask=m3)
            return 0

        lax.fori_loop(0, VREGS // 2, pass_e, 0)
        a3, b3, f3 = _scan_hist(h3, H3, lanes, a2, p)

        tau_key = jnp.where(f1, (b1 << (H2BITS + H3BITS)) | (b2 << H3BITS) | b3,
                            jnp.int32(-1))
        tau_v = jnp.full((L,), tau_key)
        tau_f = plsc.bitcast(tau_v, jnp.float32)
        tau_s = jnp.max(tau_f)
        n_eq = jnp.max(plsc.load_gather(
            h3c, [jnp.full((L,), jnp.maximum(b3, 0), jnp.int32)]))

        def bs_body(_, st):
            lo, hi = st
            mid = lax.shift_right_logical(lo + hi + 1, 1)
            c = (a3 + mid.astype(jnp.float32) * tau_s) <= p
            return jnp.where(c, mid, lo), jnp.where(c, hi, mid - 1)

        kstar, _ = lax.fori_loop(0, 17, bs_body, (jnp.int32(0), n_eq))

        def find_cut(_):
            def fc_body(i, st):
                cnt, cut = st
                key = plsc.bitcast(qbuf[pl.ds(i * L, L)], jnp.int32)
                eq = key == tau_v
                kr = cnt + plsc.cumsum(eq.astype(jnp.int32))
                hit = eq & (kr == kstar)
                cut = jnp.maximum(cut, jnp.max(jnp.where(hit, i * L + lanes,
                                                         jnp.int32(-1))))
                cnt = cnt + plsc.all_reduce_population_count(eq)
                return cnt, cut

            _, cut = lax.fori_loop(0, VREGS, fc_body,
                                   (jnp.zeros((L,), jnp.int32), jnp.int32(-1)))
            return cut

        cut = lax.cond(kstar >= n_eq, lambda _: I32MAX,
                       lambda _: lax.cond(kstar == 0, lambda __: jnp.int32(-1),
                                          find_cut, _),
                       0)
        cut_v = jnp.full((L,), cut)
        a_v = jnp.full((L,), a3)
        top1_v = jnp.full((L,), top1)

        noise_copy(1).start()

        def chunk_pair(j, carry):
            bg, bwi = carry
            for par in range(2):
                c = 2 * j + par
                noise_copy_dyn(c, nbufs[par], nsems[par]).wait()
                cbase = c * CHUNK
                nb = nbufs[par]

                @plsc.parallel_loop(0, CREGS, unroll=5, carry=(bg, bwi))
                def pass_f(i, carry, cbase=cbase, nb=nb):
                    bg, bwi = carry
                    o = i * L
                    q = qbuf[pl.ds(cbase + o, L)]
                    nz = nb[pl.ds(o, L)]
                    key = plsc.bitcast(q, jnp.int32)
                    idxv = cbase + o + lanes
                    kept = ((key > tau_v)
                            | ((key == tau_v) & (idxv <= cut_v))
                            | (idxv == top1_v))
                    gm = jnp.where(kept, q / nz, jnp.float32(-1.0))
                    upd = gm > bg
                    return jnp.where(upd, gm, bg), jnp.where(upd, idxv, bwi)

                bg, bwi = pass_f

                @pl.when(c + 2 < N_CHUNKS)
                def _():
                    noise_copy_dyn(c + 2, nbufs[par], nsems[par]).start()
            return bg, bwi

        bg, bwi = lax.fori_loop(
            0, N_CHUNKS // 2, chunk_pair,
            (jnp.full((L,), -2.0, jnp.float32), jnp.zeros((L,), jnp.int32)))

        winner = jnp.min(jnp.where(bg == jnp.max(bg), bwi, I32MAX))
        tok = jnp.where(jnp.max(t_vec) <= jnp.float32(1e-10), greedy, winner)
        toks.append(tok)

    ovec = jnp.where(lanes == 0, toks[0],
                     jnp.where(lanes == 1, toks[1], jnp.int32(0)))
    obuf[...] = ovec
    pltpu.sync_copy(obuf, out_hbm.at[wid])


@jax.jit
def _sampler(logits_flat, temps16, topp16, noise_flat):
    mesh = plsc.VectorSubcoreMesh(core_axis_name="c", subcore_axis_name="s",
                                  num_cores=2, num_subcores=16)
    k = pl.kernel(
        _sc_body,
        out_type=jax.ShapeDtypeStruct((NW, L), jnp.int32),
        mesh=mesh,
        compiler_params=pltpu.CompilerParams(needs_layout_passes=False),
        scratch_types=[
            pltpu.VMEM((V,), jnp.float32),
            pltpu.VMEM((CHUNK,), jnp.float32),
            pltpu.VMEM((CHUNK,), jnp.float32),
            pltpu.VMEM((H1,), jnp.float32),
            pltpu.VMEM((H2,), jnp.float32),
            pltpu.VMEM((H3,), jnp.float32),
            pltpu.VMEM((H3,), jnp.int32),
            pltpu.VMEM((L,), jnp.float32),
            pltpu.VMEM((L,), jnp.float32),
            pltpu.VMEM((L,), jnp.int32),
            pltpu.SemaphoreType.DMA,
            pltpu.SemaphoreType.DMA,
        ],
    )
    return k(logits_flat, temps16, topp16, noise_flat)


def kernel(logits, temperatures, top_ps):
    logits = logits.astype(jnp.float32)
    noise = _noise_flat()
    temps16 = jnp.tile(temperatures.astype(jnp.float32)[:, None], (1, L))
    topp16 = jnp.tile(top_ps.astype(jnp.float32)[:, None], (1, L))
    out = _sampler(logits.reshape(-1), temps16, topp16, noise)
    return out[:, :ROWS_PER_W].reshape(-1)


# speedup vs baseline: 33.2853x; 1.0013x over previous
import functools

import jax
import jax.numpy as jnp
from jax import lax
from jax.experimental import pallas as pl
from jax.experimental.pallas import tpu as pltpu
from jax.experimental.pallas import tpu_sc as plsc

B = 64
V = 100000
L = 16
NW = 32
ROWS_PER_W = B // NW
CHUNK = 2000
N_CHUNKS = V // CHUNK
VREGS = V // L
CREGS = CHUNK // L
H1BITS, H2BITS, H3BITS = 11, 11, 9
H1, H2, H3 = 1 << H1BITS, 1 << H2BITS, 1 << H3BITS
I32MAX = jnp.int32(2**31 - 1)

_CONSTS = {}


def _noise_flat():
    if "noise" not in _CONSTS:
        n = jax.random.exponential(jax.random.key(42), (B, V), dtype=jnp.float32)
        _CONSTS["noise"] = jnp.maximum(n, 1e-10).reshape(-1)
    return _CONSTS["noise"]


def _scan_hist(href, nbins, lanes, a0, p):
    nblk = nbins // L

    def body(j, st):
        a, bstar, found = st
        i = nblk - 1 - j
        blk = href[pl.ds(i * L, L)]
        rev = lax.rev(blk, (0,))
        incl = plsc.cumsum(rev)
        elig = (a + incl) > p
        bins_desc = i * L + (L - 1) - lanes
        cand = jnp.where(elig, bins_desc, jnp.int32(-1))
        blk_b = jnp.max(cand)
        bfound = blk_b >= 0
        inclmin = jnp.min(jnp.where(elig, incl, jnp.float32(jnp.inf)))
        safe_b = jnp.maximum(blk_b, 0)
        hb = jnp.max(plsc.load_gather(href, [jnp.full((L,), safe_b, jnp.int32)]))
        blksum = jnp.sum(blk)
        a_new = jnp.where(found, a,
                          jnp.where(bfound, a + inclmin - hb, a + blksum))
        b_new = jnp.where(found, bstar, jnp.where(bfound, blk_b, bstar))
        return a_new, b_new, found | bfound

    return lax.fori_loop(0, nblk, body, (a0, jnp.int32(0), False))


def _sc_body(logits_hbm, temps_hbm, topp_hbm, noise_hbm, out_hbm,
             qbuf, nbuf0, nbuf1, h1, h2, h3, h3c, tbuf, pbuf, obuf, sem0, sem1):
    cid = lax.axis_index("c")
    sid = lax.axis_index("s")
    wid = sid * 2 + cid
    lanes = lax.iota(jnp.int32, 16)
    nsems = [sem0, sem1]
    toks = []

    for rslot in range(ROWS_PER_W):
        row = wid * ROWS_PER_W + rslot
        pltpu.sync_copy(logits_hbm.at[pl.ds(row * V, V)], qbuf)
        pltpu.sync_copy(temps_hbm.at[row], tbuf)
        pltpu.sync_copy(topp_hbm.at[row], pbuf)

        nbufs = [nbuf0, nbuf1]

        def noise_copy(c):
            return pltpu.make_async_copy(
                noise_hbm.at[pl.ds(row * V + c * CHUNK, CHUNK)],
                nbufs[c % 2], nsems[c % 2])

        def noise_copy_dyn(c, nb, sem):
            return pltpu.make_async_copy(
                noise_hbm.at[pl.ds(row * V + c * CHUNK, CHUNK)], nb, sem)

        noise_copy(0).start()

        t_vec = tbuf[...]
        t_safe = jnp.maximum(t_vec, jnp.float32(1e-5))
        p = jnp.max(pbuf[...])

        _nsA = jax.named_scope("passA"); _nsA.__enter__()
        @plsc.parallel_loop(0, VREGS, unroll=5, carry=(
            jnp.full((L,), -jnp.inf, jnp.float32), jnp.zeros((L,), jnp.int32)))
        def pass_a(i, carry):
            bl, bi = carry
            v = qbuf[pl.ds(i * L, L)]
            upd = v > bl
            return jnp.where(upd, v, bl), jnp.where(upd, i * L + lanes, bi)

        bl, bi = pass_a
        _nsA.__exit__(None, None, None)
        m_l = jnp.max(bl)
        greedy = jnp.min(jnp.where(bl == m_l, bi, I32MAX))
        mx_vec = jnp.full((L,), m_l) / t_safe

        _nsB = jax.named_scope("passB"); _nsB.__enter__()
        @plsc.parallel_loop(0, VREGS, unroll=5,
                            carry=jnp.zeros((L,), jnp.float32))
        def pass_b(i, zacc):
            e = jnp.exp(qbuf[pl.ds(i * L, L)] / t_safe - mx_vec)
            qbuf[pl.ds(i * L, L)] = e
            return zacc + e

        zacc = pass_b
        _nsB.__exit__(None, None, None)
        z_vec = jnp.full((L,), jnp.sum(zacc))

        def clr(href, nbins, zero):
            @plsc.parallel_loop(0, nbins // L, unroll=4)
            def cb(i):
                href[pl.ds(i * L, L)] = zero

        _nsCL = jax.named_scope("clear"); _nsCL.__enter__()
        clr(h1, H1, jnp.zeros((L,), jnp.float32))
        clr(h2, H2, jnp.zeros((L,), jnp.float32))
        clr(h3, H3, jnp.zeros((L,), jnp.float32))
        clr(h3c, H3, jnp.zeros((L,), jnp.int32))

        _nsCL.__exit__(None, None, None)
        _nsC = jax.named_scope("passC"); _nsC.__enter__()
        def pass_c(i, carry):
            bq, bi2 = carry
            for u in range(2):
                o = (2 * i + u) * L
                q = qbuf[pl.ds(o, L)] / z_vec
                qbuf[pl.ds(o, L)] = q
                key = plsc.bitcast(q, jnp.int32)
                plsc.addupdate_scatter(
                    h1, [lax.shift_right_logical(key, H2BITS + H3BITS)], q)
                upd = q > bq
                bq = jnp.where(upd, q, bq)
                bi2 = jnp.where(upd, o + lanes, bi2)
            return bq, bi2

        bq, bi2 = lax.fori_loop(
            0, VREGS // 2, pass_c,
            (jnp.full((L,), -1.0, jnp.float32), jnp.zeros((L,), jnp.int32)))
        top1 = jnp.min(jnp.where(bq == jnp.max(bq), bi2, I32MAX))

        _nsC.__exit__(None, None, None)
        _nsS1 = jax.named_scope("scan1"); _nsS1.__enter__()
        a1, b1, f1 = _scan_hist(h1, H1, lanes, jnp.float32(0.0), p)

        _nsS1.__exit__(None, None, None)
        _nsD = jax.named_scope("passD"); _nsD.__enter__()
        b1v = jnp.full((L,), jnp.where(f1, b1, jnp.int32(-1)))

        def pass_d(i, _):
            for u in range(2):
                o = (2 * i + u) * L
                key = plsc.bitcast(qbuf[pl.ds(o, L)], jnp.int32)
                m2 = lax.shift_right_logical(key, H2BITS + H3BITS) == b1v
                bin2 = lax.shift_right_logical(key, H3BITS) & jnp.int32(H2 - 1)
                plsc.addupdate_scatter(h2, [bin2], qbuf[pl.ds(o, L)], mask=m2)
            return 0

        lax.fori_loop(0, VREGS // 2, pass_d, 0)
        _nsD.__exit__(None, None, None)
        a2, b2, f2 = _scan_hist(h2, H2, lanes, a1, p)

        _nsE = jax.named_scope("passE"); _nsE.__enter__()
        pre23v = jnp.full((L,), jnp.where(f1 & f2, b1 * H2 + b2, jnp.int32(-1)))
        ones_i = jnp.full((L,), 1, jnp.int32)

        def pass_e(i, _):
            for u in range(2):
                o = (2 * i + u) * L
                q = qbuf[pl.ds(o, L)]
                key = plsc.bitcast(q, jnp.int32)
                m3 = lax.shift_right_logical(key, H3BITS) == pre23v
                bin3 = key & jnp.int32(H3 - 1)
                plsc.addupdate_scatter(h3, [bin3], q, mask=m3)
                plsc.addupdate_scatter(h3c, [bin3], ones_i, mask=m3)
            return 0

        lax.fori_loop(0, VREGS // 2, pass_e, 0)
        _nsE.__exit__(None, None, None)
        a3, b3, f3 = _scan_hist(h3, H3, lanes, a2, p)

        tau_key = jnp.where(f1, (b1 << (H2BITS + H3BITS)) | (b2 << H3BITS) | b3,
                            jnp.int32(-1))
        tau_v = jnp.full((L,), tau_key)
        tau_f = plsc.bitcast(tau_v, jnp.float32)
        tau_s = jnp.max(tau_f)
        n_eq = jnp.max(plsc.load_gather(
            h3c, [jnp.full((L,), jnp.maximum(b3, 0), jnp.int32)]))

        def bs_body(_, st):
            lo, hi = st
            mid = lax.shift_right_logical(lo + hi + 1, 1)
            c = (a3 + mid.astype(jnp.float32) * tau_s) <= p
            return jnp.where(c, mid, lo), jnp.where(c, hi, mid - 1)

        kstar, _ = lax.fori_loop(0, 17, bs_body, (jnp.int32(0), n_eq))

        def find_cut(_):
            def fc_body(i, st):
                cnt, cut = st
                key = plsc.bitcast(qbuf[pl.ds(i * L, L)], jnp.int32)
                eq = key == tau_v
                kr = cnt + plsc.cumsum(eq.astype(jnp.int32))
                hit = eq & (kr == kstar)
                cut = jnp.maximum(cut, jnp.max(jnp.where(hit, i * L + lanes,
                                                         jnp.int32(-1))))
                cnt = cnt + plsc.all_reduce_population_count(eq)
                return cnt, cut

            _, cut = lax.fori_loop(0, VREGS, fc_body,
                                   (jnp.zeros((L,), jnp.int32), jnp.int32(-1)))
            return cut

        cut = lax.cond(kstar >= n_eq, lambda _: I32MAX,
                       lambda _: lax.cond(kstar == 0, lambda __: jnp.int32(-1),
                                          find_cut, _),
                       0)
        cut_v = jnp.full((L,), cut)
        a_v = jnp.full((L,), a3)
        top1_v = jnp.full((L,), top1)

        _nsF = jax.named_scope("passF"); _nsF.__enter__()
        noise_copy(1).start()

        def chunk_pair(j, carry):
            bg, bwi = carry
            for par in range(2):
                c = 2 * j + par
                noise_copy_dyn(c, nbufs[par], nsems[par]).wait()
                cbase = c * CHUNK
                nb = nbufs[par]

                @plsc.parallel_loop(0, CREGS, unroll=5, carry=(bg, bwi))
                def pass_f(i, carry, cbase=cbase, nb=nb):
                    bg, bwi = carry
                    o = i * L
                    q = qbuf[pl.ds(cbase + o, L)]
                    nz = nb[pl.ds(o, L)]
                    key = plsc.bitcast(q, jnp.int32)
                    idxv = cbase + o + lanes
                    kept = ((key > tau_v)
                            | ((key == tau_v) & (idxv <= cut_v))
                            | (idxv == top1_v))
                    gm = jnp.where(kept, q / nz, jnp.float32(-1.0))
                    upd = gm > bg
                    return jnp.where(upd, gm, bg), jnp.where(upd, idxv, bwi)

                bg, bwi = pass_f

                @pl.when(c + 2 < N_CHUNKS)
                def _():
                    noise_copy_dyn(c + 2, nbufs[par], nsems[par]).start()
            return bg, bwi

        bg, bwi = lax.fori_loop(
            0, N_CHUNKS // 2, chunk_pair,
            (jnp.full((L,), -2.0, jnp.float32), jnp.zeros((L,), jnp.int32)))

        _nsF.__exit__(None, None, None)
        winner = jnp.min(jnp.where(bg == jnp.max(bg), bwi, I32MAX))
        tok = jnp.where(jnp.max(t_vec) <= jnp.float32(1e-10), greedy, winner)
        toks.append(tok)

    ovec = jnp.where(lanes == 0, toks[0],
                     jnp.where(lanes == 1, toks[1], jnp.int32(0)))
    obuf[...] = ovec
    pltpu.sync_copy(obuf, out_hbm.at[wid])


@jax.jit
def _sampler(logits_flat, temps16, topp16, noise_flat):
    mesh = plsc.VectorSubcoreMesh(core_axis_name="c", subcore_axis_name="s",
                                  num_cores=2, num_subcores=16)
    k = pl.kernel(
        _sc_body,
        out_type=jax.ShapeDtypeStruct((NW, L), jnp.int32),
        mesh=mesh,
        compiler_params=pltpu.CompilerParams(needs_layout_passes=False),
        scratch_types=[
            pltpu.VMEM((V,), jnp.float32),
            pltpu.VMEM((CHUNK,), jnp.float32),
            pltpu.VMEM((CHUNK,), jnp.float32),
            pltpu.VMEM((H1,), jnp.float32),
            pltpu.VMEM((H2,), jnp.float32),
            pltpu.VMEM((H3,), jnp.float32),
            pltpu.VMEM((H3,), jnp.int32),
            pltpu.VMEM((L,), jnp.float32),
            pltpu.VMEM((L,), jnp.float32),
            pltpu.VMEM((L,), jnp.int32),
            pltpu.SemaphoreType.DMA,
            pltpu.SemaphoreType.DMA,
        ],
    )
    return k(logits_flat, temps16, topp16, noise_flat)


def kernel(logits, temperatures, top_ps):
    logits = logits.astype(jnp.float32)
    noise = _noise_flat()
    temps16 = jnp.tile(temperatures.astype(jnp.float32)[:, None], (1, L))
    topp16 = jnp.tile(top_ps.astype(jnp.float32)[:, None], (1, L))
    out = _sampler(logits.reshape(-1), temps16, topp16, noise)
    return out[:, :ROWS_PER_W].reshape(-1)


# speedup vs baseline: 46.6066x; 1.4002x over previous
import jax
import jax.numpy as jnp
from jax import lax
from jax.experimental import pallas as pl
from jax.experimental.pallas import tpu as pltpu
from jax.experimental.pallas import tpu_sc as plsc

B = 64
V = 100000
L = 16
NW = 32
ROWS_PER_W = B // NW
CHUNK = 2000
N_CHUNKS = V // CHUNK
VREGS = V // L
CREGS = CHUNK // L
H1BITS, H2BITS, H3BITS = 11, 11, 9
H1, H2, H3 = 1 << H1BITS, 1 << H2BITS, 1 << H3BITS
I32MAX = jnp.int32(2**31 - 1)

_CONSTS = {}


def _noise_flat():
    if "noise" not in _CONSTS:
        n = jax.random.exponential(jax.random.key(42), (B, V), dtype=jnp.float32)
        _CONSTS["noise"] = jnp.maximum(n, 1e-10).reshape(-1)
    return _CONSTS["noise"]


def _scan_hist(href, nbins, lanes, a0, p):
    nblk = nbins // L

    def body(j, st):
        a, bstar, found = st
        i = nblk - 1 - j
        blk = href[pl.ds(i * L, L)]
        rev = lax.rev(blk, (0,))
        incl = plsc.cumsum(rev)
        elig = (a + incl) > p
        bins_desc = i * L + (L - 1) - lanes
        cand = jnp.where(elig, bins_desc, jnp.int32(-1))
        blk_b = jnp.max(cand)
        bfound = blk_b >= 0
        inclmin = jnp.min(jnp.where(elig, incl, jnp.float32(jnp.inf)))
        safe_b = jnp.maximum(blk_b, 0)
        hb = jnp.max(plsc.load_gather(href, [jnp.full((L,), safe_b, jnp.int32)]))
        blksum = jnp.sum(blk)
        a_new = jnp.where(found, a,
                          jnp.where(bfound, a + inclmin - hb, a + blksum))
        b_new = jnp.where(found, bstar, jnp.where(bfound, blk_b, bstar))
        return a_new, b_new, found | bfound

    return lax.fori_loop(0, nblk, body, (a0, jnp.int32(0), False))


def _sc_body(logits_hbm, temps_hbm, topp_hbm, noise_hbm, out_hbm,
             qbuf, nbuf0, nbuf1, h1, h2, h3, h3c, tbuf, pbuf, obuf, sem0, sem1):
    cid = lax.axis_index("c")
    sid = lax.axis_index("s")
    wid = sid * 2 + cid
    lanes = lax.iota(jnp.int32, 16)
    nbufs = [nbuf0, nbuf1]
    nsems = [sem0, sem1]
    pltpu.sync_copy(temps_hbm, tbuf)
    pltpu.sync_copy(topp_hbm, pbuf)
    toks = []

    for rslot in range(ROWS_PER_W):
        row = wid * ROWS_PER_W + rslot
        pltpu.sync_copy(logits_hbm.at[row], qbuf)

        def noise_copy(c, nb, sem):
            return pltpu.make_async_copy(
                noise_hbm.at[pl.ds(row * V + c * CHUNK, CHUNK)], nb, sem)

        noise_copy(0, nbufs[0], nsems[0]).start()
        noise_copy(1, nbufs[1], nsems[1]).start()

        row_v = jnp.full((L,), row, jnp.int32)
        t_vec = plsc.load_gather(tbuf, [row_v])
        t_safe = jnp.maximum(t_vec, jnp.float32(1e-5))
        p = jnp.max(plsc.load_gather(pbuf, [row_v]))

        @plsc.parallel_loop(0, VREGS, unroll=5, carry=(
            jnp.full((L,), -jnp.inf, jnp.float32), jnp.zeros((L,), jnp.int32)))
        def pass_a(i, carry):
            bl, bi = carry
            v = qbuf[pl.ds(i * L, L)]
            upd = v > bl
            return jnp.where(upd, v, bl), jnp.where(upd, i * L + lanes, bi)

        bl, bi = pass_a
        m_l = jnp.max(bl)
        greedy = jnp.min(jnp.where(bl == m_l, bi, I32MAX))
        mx_vec = jnp.full((L,), m_l) / t_safe

        @plsc.parallel_loop(0, VREGS, unroll=5,
                            carry=jnp.zeros((L,), jnp.float32))
        def pass_b(i, zacc):
            e = jnp.exp(qbuf[pl.ds(i * L, L)] / t_safe - mx_vec)
            qbuf[pl.ds(i * L, L)] = e
            return zacc + e

        zacc = pass_b
        z_vec = jnp.full((L,), jnp.sum(zacc))

        def clr(href, nbins, zero):
            @plsc.parallel_loop(0, nbins // L, unroll=4)
            def cb(i):
                href[pl.ds(i * L, L)] = zero

        clr(h1, H1, jnp.zeros((L,), jnp.float32))
        clr(h2, H2, jnp.zeros((L,), jnp.float32))
        clr(h3, H3, jnp.zeros((L,), jnp.float32))
        clr(h3c, H3, jnp.zeros((L,), jnp.int32))

        @plsc.parallel_loop(0, VREGS, unroll=5, carry=(
            jnp.full((L,), -1.0, jnp.float32), jnp.zeros((L,), jnp.int32)))
        def pass_c(i, carry):
            bq, bi2 = carry
            q = qbuf[pl.ds(i * L, L)] / z_vec
            qbuf[pl.ds(i * L, L)] = q
            key = plsc.bitcast(q, jnp.int32)
            plsc.addupdate_scatter(
                h1, [lax.shift_right_logical(key, H2BITS + H3BITS)], q)
            upd = q > bq
            return jnp.where(upd, q, bq), jnp.where(upd, i * L + lanes, bi2)

        bq, bi2 = pass_c
        top1 = jnp.min(jnp.where(bq == jnp.max(bq), bi2, I32MAX))

        a1, b1, f1 = _scan_hist(h1, H1, lanes, jnp.float32(0.0), p)

        b1v = jnp.full((L,), jnp.where(f1, b1, jnp.int32(-1)))

        @plsc.parallel_loop(0, VREGS, unroll=5)
        def pass_d(i):
            q = qbuf[pl.ds(i * L, L)]
            key = plsc.bitcast(q, jnp.int32)
            m2 = lax.shift_right_logical(key, H2BITS + H3BITS) == b1v
            bin2 = lax.shift_right_logical(key, H3BITS) & jnp.int32(H2 - 1)
            plsc.addupdate_scatter(h2, [bin2], q, mask=m2)

        a2, b2, f2 = _scan_hist(h2, H2, lanes, a1, p)

        pre23v = jnp.full((L,), jnp.where(f1 & f2, b1 * H2 + b2, jnp.int32(-1)))
        ones_i = jnp.full((L,), 1, jnp.int32)

        @plsc.parallel_loop(0, VREGS, unroll=5)
        def pass_e(i):
            q = qbuf[pl.ds(i * L, L)]
            key = plsc.bitcast(q, jnp.int32)
            m3 = lax.shift_right_logical(key, H3BITS) == pre23v
            bin3 = key & jnp.int32(H3 - 1)
            plsc.addupdate_scatter(h3, [bin3], q, mask=m3)
            plsc.addupdate_scatter(h3c, [bin3], ones_i, mask=m3)

        a3, b3, f3 = _scan_hist(h3, H3, lanes, a2, p)

        tau_key = jnp.where(f1, (b1 << (H2BITS + H3BITS)) | (b2 << H3BITS) | b3,
                            jnp.int32(-1))
        tau_v = jnp.full((L,), tau_key)
        tau_f = plsc.bitcast(tau_v, jnp.float32)
        tau_s = jnp.max(tau_f)
        n_eq = jnp.max(plsc.load_gather(
            h3c, [jnp.full((L,), jnp.maximum(b3, 0), jnp.int32)]))

        def bs_body(_, st):
            lo, hi = st
            mid = lax.shift_right_logical(lo + hi + 1, 1)
            c = (a3 + mid.astype(jnp.float32) * tau_s) <= p
            return jnp.where(c, mid, lo), jnp.where(c, hi, mid - 1)

        kstar, _ = lax.fori_loop(0, 17, bs_body, (jnp.int32(0), n_eq))

        def find_cut(_):
            kstar_v = jnp.full((L,), kstar)

            @plsc.parallel_loop(0, VREGS, unroll=5, carry=(
                jnp.zeros((L,), jnp.int32), jnp.zeros((L,), jnp.int32),
                jnp.zeros((L,), jnp.int32)))
            def ph1(i, st):
                cnt, vloc, cbef = st
                key = plsc.bitcast(qbuf[pl.ds(i * L, L)], jnp.int32)
                eq = key == tau_v
                pc = plsc.all_reduce_population_count(eq)
                cnt2 = cnt + pc
                hit = (cnt < kstar_v) & (cnt2 >= kstar_v)
                vloc = jnp.where(hit, jnp.full((L,), i, jnp.int32), vloc)
                cbef = jnp.where(hit, cnt, cbef)
                return cnt2, vloc, cbef

            _, vloc, cbef = ph1
            vi = jnp.max(vloc)
            key = plsc.bitcast(qbuf[pl.ds(vi * L, L)], jnp.int32)
            eq = key == tau_v
            ranks = jnp.max(cbef) + plsc.cumsum(eq.astype(jnp.int32))
            hitl = eq & (ranks == kstar)
            return jnp.max(jnp.where(hitl, vi * L + lanes, jnp.int32(-1)))

        cut = lax.cond(kstar >= n_eq, lambda _: I32MAX,
                       lambda _: lax.cond(kstar == 0, lambda __: jnp.int32(-1),
                                          find_cut, _),
                       0)
        cut_v = jnp.full((L,), cut)
        top1_v = jnp.full((L,), top1)

        def chunk_pair(j, carry):
            bg, bwi = carry
            for par in range(2):
                c = 2 * j + par
                noise_copy(c, nbufs[par], nsems[par]).wait()
                cbase = c * CHUNK
                nb = nbufs[par]

                @plsc.parallel_loop(0, CREGS, unroll=5, carry=(bg, bwi))
                def pass_f(i, carry, cbase=cbase, nb=nb):
                    bg, bwi = carry
                    o = i * L
                    q = qbuf[pl.ds(cbase + o, L)]
                    nz = nb[pl.ds(o, L)]
                    key = plsc.bitcast(q, jnp.int32)
                    idxv = cbase + o + lanes
                    kept = ((key > tau_v)
                            | ((key == tau_v) & (idxv <= cut_v))
                            | (idxv == top1_v))
                    gm = jnp.where(kept, q / nz, jnp.float32(-1.0))
                    upd = gm > bg
                    return jnp.where(upd, gm, bg), jnp.where(upd, idxv, bwi)

                bg, bwi = pass_f

                @pl.when(c + 2 < N_CHUNKS)
                def _():
                    noise_copy(c + 2, nbufs[par], nsems[par]).start()
            return bg, bwi

        bg, bwi = lax.fori_loop(
            0, N_CHUNKS // 2, chunk_pair,
            (jnp.full((L,), -2.0, jnp.float32), jnp.zeros((L,), jnp.int32)))

        winner = jnp.min(jnp.where(bg == jnp.max(bg), bwi, I32MAX))
        tok = jnp.where(jnp.max(t_vec) <= jnp.float32(1e-10), greedy, winner)
        toks.append(tok)

    ovec = jnp.where(lanes == 0, toks[0],
                     jnp.where(lanes == 1, toks[1], jnp.int32(0)))
    obuf[...] = ovec
    pltpu.sync_copy(obuf, out_hbm.at[wid])


@jax.jit
def _sampler(logits, temps, topps, noise_flat):
    mesh = plsc.VectorSubcoreMesh(core_axis_name="c", subcore_axis_name="s",
                                  num_cores=2, num_subcores=16)
    k = pl.kernel(
        _sc_body,
        out_type=jax.ShapeDtypeStruct((NW, L), jnp.int32),
        mesh=mesh,
        compiler_params=pltpu.CompilerParams(needs_layout_passes=False),
        scratch_types=[
            pltpu.VMEM((V,), jnp.float32),
            pltpu.VMEM((CHUNK,), jnp.float32),
            pltpu.VMEM((CHUNK,), jnp.float32),
            pltpu.VMEM((H1,), jnp.float32),
            pltpu.VMEM((H2,), jnp.float32),
            pltpu.VMEM((H3,), jnp.float32),
            pltpu.VMEM((H3,), jnp.int32),
            pltpu.VMEM((B,), jnp.float32),
            pltpu.VMEM((B,), jnp.float32),
            pltpu.VMEM((L,), jnp.int32),
            pltpu.SemaphoreType.DMA,
            pltpu.SemaphoreType.DMA,
        ],
    )
    return k(logits, temps, topps, noise_flat)


def kernel(logits, temperatures, top_ps):
    logits = logits.astype(jnp.float32)
    noise = _noise_flat()
    out = _sampler(logits, temperatures.astype(jnp.float32),
                   top_ps.astype(jnp.float32), noise)
    return out[:, :ROWS_PER_W].reshape(-1)
